# baseline (speedup 1.0000x reference)
import jax
import jax.numpy as jnp
from jax.experimental import pallas as pl

_SIZE = 32000
_SMOOTHING = 0.1
_CONFIDENCE = 1.0 - _SMOOTHING
_EPS = _SMOOTHING / (_SIZE - 2)

_BR = 64
_BC = 32000


def _fill_block(t_ref, o_ref):
    t = t_ref[0, 0, :]
    cols = jax.lax.broadcasted_iota(jnp.int32, (_BR, _BC), 1)
    o_ref[...] = jnp.where(cols == t[:, None],
                           jnp.float32(_CONFIDENCE), jnp.float32(_EPS))


def kernel(target):
    n = target.shape[0]
    t3 = target.astype(jnp.int32).reshape(n // _BR, 1, _BR)
    out = pl.pallas_call(
        _fill_block,
        grid=(n // _BR,),
        in_specs=[pl.BlockSpec((1, 1, _BR), lambda i: (i, 0, 0))],
        out_specs=pl.BlockSpec((_BR, _BC), lambda i: (i, 0)),
        out_shape=jax.ShapeDtypeStruct((n, _SIZE), jnp.float32),
    )(t3)
    return out


# speedup vs baseline: 1.0035x; 1.0035x over previous
import jax
import jax.numpy as jnp
from jax.experimental import pallas as pl

_SIZE = 32000
_SMOOTHING = 0.1
_CONFIDENCE = 1.0 - _SMOOTHING
_EPS = _SMOOTHING / (_SIZE - 2)

_BR = 128
_BC = 32000


def _fill_block(t_ref, o_ref):
    t = t_ref[0, 0, :]
    cols = jax.lax.broadcasted_iota(jnp.int32, (_BR, _BC), 1)
    o_ref[...] = jnp.where(cols == t[:, None],
                           jnp.float32(_CONFIDENCE), jnp.float32(_EPS))


def kernel(target):
    n = target.shape[0]
    t3 = target.astype(jnp.int32).reshape(n // _BR, 1, _BR)
    out = pl.pallas_call(
        _fill_block,
        grid=(n // _BR,),
        in_specs=[pl.BlockSpec((1, 1, _BR), lambda i: (i, 0, 0))],
        out_specs=pl.BlockSpec((_BR, _BC), lambda i: (i, 0)),
        out_shape=jax.ShapeDtypeStruct((n, _SIZE), jnp.float32),
    )(t3)
    return out
